# baseline (speedup 1.0000x reference)
import functools

import jax
import jax.numpy as jnp
from jax.experimental import pallas as pl

NUM_MODULES = 16
TOPK = 2
HID = 128
D0 = 4096
B = 8192
NEG = -1e9


def _top2_selectors(e, n_rows):
    rows = jax.lax.broadcasted_iota(jnp.int32, e.shape, 0)
    mx = jnp.max(e, axis=0)
    idx = jnp.min(jnp.where(e == mx[None, :], rows, n_rows), axis=0)
    hard0 = rows == idx[None, :]
    work = jnp.where(hard0, NEG, e)
    mx2 = jnp.max(work, axis=0)
    idx2 = jnp.min(jnp.where(work == mx2[None, :], rows, n_rows), axis=0)
    hard1 = rows == idx2[None, :]
    return jnp.concatenate([hard0.astype(jnp.float32),
                            hard1.astype(jnp.float32)], axis=1)


def _selector_kernel(e0_ref, e1_ref, eo_ref, s0_ref, s1_ref, so_ref):
    s0_ref[...] = _top2_selectors(e0_ref[...], D0)
    s1_ref[...] = _top2_selectors(e1_ref[...], NUM_MODULES)
    so_ref[...] = _top2_selectors(eo_ref[...], NUM_MODULES)


def _mlp_kernel(x_ref, s0_ref, w1a_ref, b1a_ref, w2a_ref, b2a_ref,
                s1_ref, w1b_ref, b1b_ref, w2b_ref, b2b_ref, so_ref, out_ref):
    f32 = jnp.float32
    x = x_ref[...]
    g = jnp.dot(x, s0_ref[...], preferred_element_type=f32)
    hid = jax.nn.relu(jnp.dot(g, w1a_ref[...], preferred_element_type=f32)
                      + b1a_ref[...])
    h1 = jnp.dot(hid, w2a_ref[...], preferred_element_type=f32) + b2a_ref[...]
    g1 = jnp.dot(h1, s1_ref[...], preferred_element_type=f32)
    hid2 = jax.nn.relu(jnp.dot(g1, w1b_ref[...], preferred_element_type=f32)
                       + b1b_ref[...])
    h2 = jnp.dot(hid2, w2b_ref[...], preferred_element_type=f32) + b2b_ref[...]
    v = jnp.dot(h2, so_ref[...], preferred_element_type=f32)
    out_ref[...] = jax.nn.sigmoid(v)


def _pack_layer(W1, b1, W2, b2):
    M, K, H = W1.shape
    eye = jnp.eye(M, dtype=W1.dtype)
    w1blk = (W1.transpose(1, 0, 2)[:, :, None, :] * eye[None, :, :, None]
             ).reshape(K * M, M * H)
    w2blk = (W2[:, :, 0][:, :, None] * eye[:, None, :]).reshape(M * H, M)
    return w1blk, b1.reshape(1, M * H), w2blk, b2.reshape(1, M)


def kernel(x, task_id, emb0, emb1, emb_out, W1_0, b1_0, W2_0, b2_0,
           W1_1, b1_1, W2_1, b2_1):
    del task_id
    f32 = jnp.float32
    e0 = emb0[0]
    e1 = emb1[0]
    eo = emb_out[0]

    s0, s1, so = pl.pallas_call(
        _selector_kernel,
        out_shape=(
            jax.ShapeDtypeStruct((D0, 2 * NUM_MODULES), f32),
            jax.ShapeDtypeStruct((NUM_MODULES, 2 * NUM_MODULES), f32),
            jax.ShapeDtypeStruct((NUM_MODULES, 2), f32),
        ),
    )(e0, e1, eo)

    w1a, b1a, w2a, b2a = _pack_layer(W1_0, b1_0, W2_0, b2_0)
    w1b, b1b, w2b, b2b = _pack_layer(W1_1, b1_1, W2_1, b2_1)

    TB = 512
    grid = (B // TB,)
    out = pl.pallas_call(
        _mlp_kernel,
        grid=grid,
        in_specs=[
            pl.BlockSpec((TB, D0), lambda i: (i, 0)),
            pl.BlockSpec((D0, 32), lambda i: (0, 0)),
            pl.BlockSpec((32, NUM_MODULES * HID), lambda i: (0, 0)),
            pl.BlockSpec((1, NUM_MODULES * HID), lambda i: (0, 0)),
            pl.BlockSpec((NUM_MODULES * HID, NUM_MODULES), lambda i: (0, 0)),
            pl.BlockSpec((1, NUM_MODULES), lambda i: (0, 0)),
            pl.BlockSpec((NUM_MODULES, 32), lambda i: (0, 0)),
            pl.BlockSpec((32, NUM_MODULES * HID), lambda i: (0, 0)),
            pl.BlockSpec((1, NUM_MODULES * HID), lambda i: (0, 0)),
            pl.BlockSpec((NUM_MODULES * HID, NUM_MODULES), lambda i: (0, 0)),
            pl.BlockSpec((1, NUM_MODULES), lambda i: (0, 0)),
            pl.BlockSpec((NUM_MODULES, 2), lambda i: (0, 0)),
        ],
        out_specs=pl.BlockSpec((TB, 2), lambda i: (i, 0)),
        out_shape=jax.ShapeDtypeStruct((B, 2), f32),
    )(x, s0, w1a, b1a, w2a, b2a, s1, w1b, b1b, w2b, b2b, so)
    return out
